# Initial kernel scaffold; baseline (speedup 1.0000x reference)
import jax
import jax.numpy as jnp
from jax.experimental import pallas as pl


def kernel(indices, targets, embedding):
    raise NotImplementedError("write your pallas kernel here")


# speedup vs baseline: 1.3743x; 1.3743x over previous
import functools

import jax
import jax.numpy as jnp
from jax import lax
from jax.experimental import pallas as pl
from jax.experimental.pallas import tpu as pltpu
from jax.experimental.pallas import tpu_sc as plsc

_VOCAB = 1000
_N = 51200
_NC = 2
_NS = 16
_NW = _NC * _NS
_PER_W = _N // _NW
_C = 64
_NCHUNK = _PER_W // _C


def _lse_body(emb_ref, out_ref):
    x = emb_ref[...]
    m = jnp.max(x, axis=1, keepdims=True)
    s = jnp.sum(jnp.exp(x - m), axis=1, keepdims=True)
    out_ref[...] = jnp.log(s) + m


def _compute_lse(embedding):
    out = pl.pallas_call(
        _lse_body,
        out_shape=jax.ShapeDtypeStruct((_VOCAB, 1), jnp.float32),
    )(embedding)
    return out.reshape(_VOCAB)


_sc_mesh = plsc.VectorSubcoreMesh(core_axis_name="c", subcore_axis_name="s")


@functools.partial(
    pl.kernel,
    out_type=(
        jax.ShapeDtypeStruct((_N, _VOCAB), jnp.float32),
        jax.ShapeDtypeStruct((_N,), jnp.float32),
    ),
    mesh=_sc_mesh,
    compiler_params=pltpu.CompilerParams(
        use_tc_tiling_on_sc=False, needs_layout_passes=False),
    scratch_types=[
        pltpu.VMEM((_C,), jnp.int32),
        pltpu.VMEM((_C,), jnp.int32),
        pltpu.VMEM((_C, _VOCAB), jnp.float32),
        pltpu.VMEM((_C,), jnp.float32),
        pltpu.VMEM((_VOCAB,), jnp.float32),
        pltpu.SemaphoreType.DMA,
    ],
)
def _sc_gather_loss(emb_hbm, idx_hbm, tgt_hbm, lse_hbm, out_hbm, loss_hbm,
                    idx_v, tgt_v, rows_v, loss_v, lse_v, sem):
    wid = lax.axis_index("s") * _NC + lax.axis_index("c")
    base_w = wid * _PER_W
    pltpu.sync_copy(lse_hbm, lse_v)

    def chunk(c, carry):
        base = pl.multiple_of(base_w + c * _C, _C)
        pltpu.sync_copy(idx_hbm.at[pl.ds(base, _C)], idx_v)
        pltpu.sync_copy(tgt_hbm.at[pl.ds(base, _C)], tgt_v)
        pltpu.async_copy(emb_hbm.at[idx_v], rows_v, sem).wait()
        for j in range(_C // 16):
            sl = pl.ds(j * 16, 16)
            idx16 = idx_v[sl]
            tgt16 = tgt_v[sl]
            row16 = lax.iota(jnp.int32, 16) + (j * 16)
            true_logit = plsc.load_gather(rows_v, [row16, tgt16])
            lse16 = plsc.load_gather(lse_v, [idx16])
            loss_v[sl] = lse16 - true_logit
        pltpu.sync_copy(rows_v, out_hbm.at[pl.ds(base, _C)])
        pltpu.sync_copy(loss_v, loss_hbm.at[pl.ds(base, _C)])
        return carry

    lax.fori_loop(0, _NCHUNK, chunk, 0)


def kernel(indices, targets, embedding):
    idx_flat = indices.reshape(_N)
    tgt_flat = targets.reshape(_N)
    lse = _compute_lse(embedding)
    logits_flat, loss = _sc_gather_loss(embedding, idx_flat, tgt_flat, lse)
    B, L = indices.shape
    return logits_flat.reshape(B, L, _VOCAB), loss


# speedup vs baseline: 1.4194x; 1.0328x over previous
import functools

import jax
import jax.numpy as jnp
from jax import lax
from jax.experimental import pallas as pl
from jax.experimental.pallas import tpu as pltpu
from jax.experimental.pallas import tpu_sc as plsc

_VOCAB = 1000
_N = 51200
_NC = 2
_NS = 16
_NW = _NC * _NS
_PER_W = _N // _NW
_C = 32
_NCHUNK = _PER_W // _C


def _lse_body(emb_ref, out_ref):
    x = emb_ref[...]
    m = jnp.max(x, axis=1, keepdims=True)
    s = jnp.sum(jnp.exp(x - m), axis=1, keepdims=True)
    out_ref[...] = jnp.log(s) + m


def _compute_lse(embedding):
    out = pl.pallas_call(
        _lse_body,
        out_shape=jax.ShapeDtypeStruct((_VOCAB, 1), jnp.float32),
    )(embedding)
    return out.reshape(_VOCAB)


_sc_mesh = plsc.VectorSubcoreMesh(core_axis_name="c", subcore_axis_name="s")


@functools.partial(
    pl.kernel,
    out_type=(
        jax.ShapeDtypeStruct((_N, _VOCAB), jnp.float32),
        jax.ShapeDtypeStruct((_N,), jnp.float32),
    ),
    mesh=_sc_mesh,
    compiler_params=pltpu.CompilerParams(
        use_tc_tiling_on_sc=False, needs_layout_passes=False),
    scratch_types=[
        pltpu.VMEM((_PER_W,), jnp.int32),
        pltpu.VMEM((_PER_W,), jnp.int32),
        pltpu.VMEM((_C, _VOCAB), jnp.float32),
        pltpu.VMEM((_C, _VOCAB), jnp.float32),
        pltpu.VMEM((_PER_W,), jnp.float32),
        pltpu.VMEM((_VOCAB,), jnp.float32),
        pltpu.SemaphoreType.DMA,
        pltpu.SemaphoreType.DMA,
        pltpu.SemaphoreType.DMA,
        pltpu.SemaphoreType.DMA,
    ],
)
def _sc_gather_loss(emb_hbm, idx_hbm, tgt_hbm, lse_hbm, out_hbm, loss_hbm,
                    idx_all, tgt_all, rows0, rows1, loss_all, lse_v,
                    sem_g0, sem_g1, sem_w0, sem_w1):
    wid = lax.axis_index("s") * _NC + lax.axis_index("c")
    base_w = wid * _PER_W
    pltpu.sync_copy(idx_hbm.at[pl.ds(base_w, _PER_W)], idx_all)
    pltpu.sync_copy(tgt_hbm.at[pl.ds(base_w, _PER_W)], tgt_all)
    pltpu.sync_copy(lse_hbm, lse_v)

    def issue_gather(c, buf, sem):
        pltpu.async_copy(emb_hbm.at[idx_all.at[pl.ds(c * _C, _C)]], buf, sem)

    def wait_gather(buf, sem):
        pltpu.make_async_copy(emb_hbm.at[idx_all.at[pl.ds(0, _C)]], buf,
                              sem).wait()

    def process(c, buf, wsem):
        local = pl.multiple_of(c * _C, _C)
        for j in range(_C // 16):
            sl = pl.ds(local + j * 16, 16)
            idx16 = idx_all[sl]
            tgt16 = tgt_all[sl]
            row16 = lax.iota(jnp.int32, 16) + (j * 16)
            true_logit = plsc.load_gather(buf, [row16, tgt16])
            lse16 = plsc.load_gather(lse_v, [idx16])
            loss_all[sl] = lse16 - true_logit
        pltpu.async_copy(buf, out_hbm.at[pl.ds(base_w + local, _C)], wsem)

    def wait_write(buf, sem):
        pltpu.make_async_copy(buf, out_hbm.at[pl.ds(base_w, _C)], sem).wait()

    issue_gather(0, rows0, sem_g0)
    issue_gather(1, rows1, sem_g1)

    def step(i, carry):
        c0 = i * 2
        wait_gather(rows0, sem_g0)
        process(c0, rows0, sem_w0)
        wait_gather(rows1, sem_g1)
        process(c0 + 1, rows1, sem_w1)
        wait_write(rows0, sem_w0)
        issue_gather(c0 + 2, rows0, sem_g0)
        wait_write(rows1, sem_w1)
        issue_gather(c0 + 3, rows1, sem_g1)
        return carry

    lax.fori_loop(0, _NCHUNK // 2 - 1, step, 0)

    wait_gather(rows0, sem_g0)
    process(_NCHUNK - 2, rows0, sem_w0)
    wait_gather(rows1, sem_g1)
    process(_NCHUNK - 1, rows1, sem_w1)
    wait_write(rows0, sem_w0)
    wait_write(rows1, sem_w1)
    pltpu.sync_copy(loss_all, loss_hbm.at[pl.ds(base_w, _PER_W)])


def kernel(indices, targets, embedding):
    idx_flat = indices.reshape(_N)
    tgt_flat = targets.reshape(_N)
    lse = _compute_lse(embedding)
    logits_flat, loss = _sc_gather_loss(embedding, idx_flat, tgt_flat, lse)
    B, L = indices.shape
    return logits_flat.reshape(B, L, _VOCAB), loss
